# Initial kernel scaffold; baseline (speedup 1.0000x reference)
import jax
import jax.numpy as jnp
from jax.experimental import pallas as pl


def kernel(src_tok_idxs, emb_table, W, b):
    raise NotImplementedError("write your pallas kernel here")


# speedup vs baseline: 11.1664x; 11.1664x over previous
import math

import jax
import jax.numpy as jnp
import numpy as np
from jax import lax
from jax.experimental import pallas as pl
from jax.experimental.pallas import tpu as pltpu
from jax.experimental.pallas import tpu_sc as plsc

VOCAB = 100000
EMBED = 64
OUT_DIM = 128
B = 4096
L = 200
MAX_LEN = 200

NUM_CORES = 2
NUM_SUBCORES = 16
NW = NUM_CORES * NUM_SUBCORES
B_PER_W = B // NW
G0, G1 = 96, 104


def _positional_table():
    position = np.arange(MAX_LEN, dtype=np.float32)[:, None]
    div_term = np.exp(
        np.arange(0, EMBED, 2, dtype=np.float32) * (-math.log(10000.0) / EMBED))
    pe = np.zeros((MAX_LEN, EMBED), dtype=np.float32)
    pe[:, 0::2] = np.sin(position * div_term)
    pe[:, 1::2] = np.cos(position * div_term)
    return pe


_PE = _positional_table()


def _sc_gather_sum(emb_table, idx_flat):
    mesh = plsc.VectorSubcoreMesh(core_axis_name="c", subcore_axis_name="s")

    def body(table_hbm, idx_hbm, out_hbm, idx_v, rows_v, out_v, sem):
        wid = lax.axis_index("s") * NUM_CORES + lax.axis_index("c")
        base = wid * (B_PER_W * L)
        pltpu.sync_copy(idx_hbm.at[pl.ds(base, B_PER_W * L)], idx_v)

        def seq_body(r, carry):
            off = r * L
            cp0 = pltpu.make_async_copy(
                table_hbm.at[idx_v.at[pl.ds(off, G0)]],
                rows_v.at[pl.ds(0, G0)], sem)
            cp0.start()
            cp1 = pltpu.make_async_copy(
                table_hbm.at[idx_v.at[pl.ds(off + G0, G1)]],
                rows_v.at[pl.ds(G0, G1)], sem)
            cp1.start()
            cp0.wait()
            cp1.wait()

            def red(l, accs):
                a0, a1, a2, a3 = accs
                a0 = a0 + rows_v[l, pl.ds(0, 16)]
                a1 = a1 + rows_v[l, pl.ds(16, 16)]
                a2 = a2 + rows_v[l, pl.ds(32, 16)]
                a3 = a3 + rows_v[l, pl.ds(48, 16)]
                return (a0, a1, a2, a3)

            z = jnp.zeros((16,), jnp.float32)
            a0, a1, a2, a3 = lax.fori_loop(0, L, red, (z, z, z, z))
            out_v[r, pl.ds(0, 16)] = a0
            out_v[r, pl.ds(16, 16)] = a1
            out_v[r, pl.ds(32, 16)] = a2
            out_v[r, pl.ds(48, 16)] = a3
            return carry

        lax.fori_loop(0, B_PER_W, seq_body, 0)
        pltpu.sync_copy(out_v, out_hbm.at[pl.ds(wid * B_PER_W, B_PER_W)])

    call = pl.kernel(
        body,
        out_type=jax.ShapeDtypeStruct((B, EMBED), jnp.float32),
        mesh=mesh,
        compiler_params=pltpu.CompilerParams(use_tc_tiling_on_sc=False),
        scratch_types=[
            pltpu.VMEM((B_PER_W * L,), jnp.int32),
            pltpu.VMEM((L, EMBED), jnp.float32),
            pltpu.VMEM((B_PER_W, EMBED), jnp.float32),
            pltpu.SemaphoreType.DMA,
        ],
    )
    return call(emb_table, idx_flat)


def _tc_combine_body(idx_ref, sums_ref, pe_ref, wt_ref, b_ref, e1_ref, out_ref):
    idx = idx_ref[...]
    m = jnp.logical_and(idx != 0, idx != 1).astype(jnp.float32)
    n_eos = jnp.sum((idx == 1).astype(jnp.float32), axis=1, keepdims=True)
    count = jnp.sum(m, axis=1, keepdims=True)
    pe_sum = jnp.dot(m, pe_ref[...], preferred_element_type=jnp.float32)
    s_masked = sums_ref[...] - n_eos * e1_ref[...]
    pooled = (8.0 * s_masked + pe_sum) / (count + 1e-6)
    out_ref[...] = (
        jnp.dot(pooled, wt_ref[...], preferred_element_type=jnp.float32)
        + b_ref[...])


def _tc_combine(idx, sums, wt, bvec, e1):
    return pl.pallas_call(
        _tc_combine_body,
        out_shape=jax.ShapeDtypeStruct((B, OUT_DIM), jnp.float32),
    )(idx, sums, jnp.asarray(_PE), wt, bvec, e1)


def kernel(src_tok_idxs, emb_table, W, b):
    idx = src_tok_idxs.astype(jnp.int32)
    sums = _sc_gather_sum(emb_table, idx.reshape(-1))
    return _tc_combine(idx, sums, W.T, b.reshape(1, OUT_DIM),
                       emb_table[1:2])


# speedup vs baseline: 18.1141x; 1.6222x over previous
import math

import jax
import jax.numpy as jnp
import numpy as np
from jax import lax
from jax.experimental import pallas as pl
from jax.experimental.pallas import tpu as pltpu
from jax.experimental.pallas import tpu_sc as plsc

VOCAB = 100000
EMBED = 64
OUT_DIM = 128
B = 4096
L = 200
MAX_LEN = 200

NUM_CORES = 2
NUM_SUBCORES = 16
NW = NUM_CORES * NUM_SUBCORES
B_PER_W = B // NW
G0, G1 = 96, 104


def _positional_table():
    position = np.arange(MAX_LEN, dtype=np.float32)[:, None]
    div_term = np.exp(
        np.arange(0, EMBED, 2, dtype=np.float32) * (-math.log(10000.0) / EMBED))
    pe = np.zeros((MAX_LEN, EMBED), dtype=np.float32)
    pe[:, 0::2] = np.sin(position * div_term)
    pe[:, 1::2] = np.cos(position * div_term)
    return pe


_PE = _positional_table()


def _sc_gather_sum(emb_table, idx_flat):
    mesh = plsc.VectorSubcoreMesh(core_axis_name="c", subcore_axis_name="s")

    def body(table_hbm, idx_hbm, out_hbm, idx_v, rows_v, out_v, sem0, sem1):
        wid = lax.axis_index("s") * NUM_CORES + lax.axis_index("c")
        base = wid * (B_PER_W * L)
        pltpu.sync_copy(idx_hbm.at[pl.ds(base, B_PER_W * L)], idx_v)

        def gather(slot, off, sem):
            pltpu.make_async_copy(
                table_hbm.at[idx_v.at[pl.ds(off, G0)]],
                rows_v.at[slot, pl.ds(0, G0)], sem).start()
            pltpu.make_async_copy(
                table_hbm.at[idx_v.at[pl.ds(off + G0, G1)]],
                rows_v.at[slot, pl.ds(G0, G1)], sem).start()

        def drain(slot, sem):
            pltpu.make_async_copy(
                table_hbm.at[idx_v.at[pl.ds(0, G0)]],
                rows_v.at[slot, pl.ds(0, G0)], sem).wait()
            pltpu.make_async_copy(
                table_hbm.at[idx_v.at[pl.ds(0, G1)]],
                rows_v.at[slot, pl.ds(G0, G1)], sem).wait()

        def reduce_into(slot, r):
            def red(l, accs):
                b0, b1, b2, b3, c0, c1, c2, c3 = accs
                row = 4 * l
                b0 = b0 + rows_v[slot, row, pl.ds(0, 16)]
                b1 = b1 + rows_v[slot, row, pl.ds(16, 16)]
                b2 = b2 + rows_v[slot, row, pl.ds(32, 16)]
                b3 = b3 + rows_v[slot, row, pl.ds(48, 16)]
                c0 = c0 + rows_v[slot, row + 1, pl.ds(0, 16)]
                c1 = c1 + rows_v[slot, row + 1, pl.ds(16, 16)]
                c2 = c2 + rows_v[slot, row + 1, pl.ds(32, 16)]
                c3 = c3 + rows_v[slot, row + 1, pl.ds(48, 16)]
                b0 = b0 + rows_v[slot, row + 2, pl.ds(0, 16)]
                b1 = b1 + rows_v[slot, row + 2, pl.ds(16, 16)]
                b2 = b2 + rows_v[slot, row + 2, pl.ds(32, 16)]
                b3 = b3 + rows_v[slot, row + 2, pl.ds(48, 16)]
                c0 = c0 + rows_v[slot, row + 3, pl.ds(0, 16)]
                c1 = c1 + rows_v[slot, row + 3, pl.ds(16, 16)]
                c2 = c2 + rows_v[slot, row + 3, pl.ds(32, 16)]
                c3 = c3 + rows_v[slot, row + 3, pl.ds(48, 16)]
                return (b0, b1, b2, b3, c0, c1, c2, c3)

            z = jnp.zeros((16,), jnp.float32)
            accs = lax.fori_loop(0, L // 4, red, (z,) * 8)
            out_v[r, pl.ds(0, 16)] = accs[0] + accs[4]
            out_v[r, pl.ds(16, 16)] = accs[1] + accs[5]
            out_v[r, pl.ds(32, 16)] = accs[2] + accs[6]
            out_v[r, pl.ds(48, 16)] = accs[3] + accs[7]

        gather(0, 0, sem0)

        def pair_body(i, carry):
            r0 = 2 * i
            gather(1, (r0 + 1) * L, sem1)
            drain(0, sem0)
            reduce_into(0, r0)

            @pl.when(r0 + 2 < B_PER_W)
            def _():
                gather(0, (r0 + 2) * L, sem0)

            drain(1, sem1)
            reduce_into(1, r0 + 1)
            return carry

        lax.fori_loop(0, B_PER_W // 2, pair_body, 0)
        pltpu.sync_copy(out_v, out_hbm.at[pl.ds(wid * B_PER_W, B_PER_W)])

    call = pl.kernel(
        body,
        out_type=jax.ShapeDtypeStruct((B, EMBED), jnp.float32),
        mesh=mesh,
        compiler_params=pltpu.CompilerParams(use_tc_tiling_on_sc=False),
        scratch_types=[
            pltpu.VMEM((B_PER_W * L,), jnp.int32),
            pltpu.VMEM((2, L, EMBED), jnp.float32),
            pltpu.VMEM((B_PER_W, EMBED), jnp.float32),
            pltpu.SemaphoreType.DMA,
            pltpu.SemaphoreType.DMA,
        ],
    )
    return call(emb_table, idx_flat)


def _tc_combine_body(idx_ref, sums_ref, pe_ref, wt_ref, b_ref, e1_ref, out_ref):
    idx = idx_ref[...]
    m = jnp.logical_and(idx != 0, idx != 1).astype(jnp.float32)
    n_eos = jnp.sum((idx == 1).astype(jnp.float32), axis=1, keepdims=True)
    count = jnp.sum(m, axis=1, keepdims=True)
    pe_sum = jnp.dot(m, pe_ref[...], preferred_element_type=jnp.float32)
    s_masked = sums_ref[...] - n_eos * e1_ref[...]
    pooled = (8.0 * s_masked + pe_sum) / (count + 1e-6)
    out_ref[...] = (
        jnp.dot(pooled, wt_ref[...], preferred_element_type=jnp.float32)
        + b_ref[...])


def _tc_combine(idx, sums, wt, bvec, e1):
    return pl.pallas_call(
        _tc_combine_body,
        out_shape=jax.ShapeDtypeStruct((B, OUT_DIM), jnp.float32),
    )(idx, sums, jnp.asarray(_PE), wt, bvec, e1)


def kernel(src_tok_idxs, emb_table, W, b):
    idx = src_tok_idxs.astype(jnp.int32)
    sums = _sc_gather_sum(emb_table, idx.reshape(-1))
    return _tc_combine(idx, sums, W.T, b.reshape(1, OUT_DIM),
                       emb_table[1:2])


# speedup vs baseline: 18.1556x; 1.0023x over previous
import math

import jax
import jax.numpy as jnp
import numpy as np
from jax import lax
from jax.experimental import pallas as pl
from jax.experimental.pallas import tpu as pltpu
from jax.experimental.pallas import tpu_sc as plsc

VOCAB = 100000
EMBED = 64
OUT_DIM = 128
B = 4096
L = 200
MAX_LEN = 200

NUM_CORES = 2
NUM_SUBCORES = 16
NW = NUM_CORES * NUM_SUBCORES
B_PER_W = B // NW
G0, G1 = 96, 104


def _positional_table():
    position = np.arange(MAX_LEN, dtype=np.float32)[:, None]
    div_term = np.exp(
        np.arange(0, EMBED, 2, dtype=np.float32) * (-math.log(10000.0) / EMBED))
    pe = np.zeros((MAX_LEN, EMBED), dtype=np.float32)
    pe[:, 0::2] = np.sin(position * div_term)
    pe[:, 1::2] = np.cos(position * div_term)
    return pe


_PE = _positional_table()


def _sc_gather_sum(emb_table, idx):
    mesh = plsc.VectorSubcoreMesh(core_axis_name="c", subcore_axis_name="s")

    def body(table_hbm, idx_hbm, out_hbm, idx_v, rows_v, out_v, sem0, sem1):
        wid = lax.axis_index("s") * NUM_CORES + lax.axis_index("c")
        pltpu.sync_copy(idx_hbm.at[pl.ds(wid * B_PER_W, B_PER_W)], idx_v)

        def gather(slot, r, sem):
            pltpu.make_async_copy(
                table_hbm.at[idx_v.at[r, pl.ds(0, G0)]],
                rows_v.at[slot, pl.ds(0, G0)], sem).start()
            pltpu.make_async_copy(
                table_hbm.at[idx_v.at[r, pl.ds(G0, G1)]],
                rows_v.at[slot, pl.ds(G0, G1)], sem).start()

        def drain(slot, sem):
            pltpu.make_async_copy(
                table_hbm.at[idx_v.at[0, pl.ds(0, G0)]],
                rows_v.at[slot, pl.ds(0, G0)], sem).wait()
            pltpu.make_async_copy(
                table_hbm.at[idx_v.at[0, pl.ds(G0, G1)]],
                rows_v.at[slot, pl.ds(G0, G1)], sem).wait()

        def reduce_into(slot, r):
            def red(l, accs):
                b0, b1, b2, b3, c0, c1, c2, c3 = accs
                row = 4 * l
                b0 = b0 + rows_v[slot, row, pl.ds(0, 16)]
                b1 = b1 + rows_v[slot, row, pl.ds(16, 16)]
                b2 = b2 + rows_v[slot, row, pl.ds(32, 16)]
                b3 = b3 + rows_v[slot, row, pl.ds(48, 16)]
                c0 = c0 + rows_v[slot, row + 1, pl.ds(0, 16)]
                c1 = c1 + rows_v[slot, row + 1, pl.ds(16, 16)]
                c2 = c2 + rows_v[slot, row + 1, pl.ds(32, 16)]
                c3 = c3 + rows_v[slot, row + 1, pl.ds(48, 16)]
                b0 = b0 + rows_v[slot, row + 2, pl.ds(0, 16)]
                b1 = b1 + rows_v[slot, row + 2, pl.ds(16, 16)]
                b2 = b2 + rows_v[slot, row + 2, pl.ds(32, 16)]
                b3 = b3 + rows_v[slot, row + 2, pl.ds(48, 16)]
                c0 = c0 + rows_v[slot, row + 3, pl.ds(0, 16)]
                c1 = c1 + rows_v[slot, row + 3, pl.ds(16, 16)]
                c2 = c2 + rows_v[slot, row + 3, pl.ds(32, 16)]
                c3 = c3 + rows_v[slot, row + 3, pl.ds(48, 16)]
                return (b0, b1, b2, b3, c0, c1, c2, c3)

            z = jnp.zeros((16,), jnp.float32)
            accs = lax.fori_loop(0, L // 4, red, (z,) * 8)
            out_v[r, pl.ds(0, 16)] = accs[0] + accs[4]
            out_v[r, pl.ds(16, 16)] = accs[1] + accs[5]
            out_v[r, pl.ds(32, 16)] = accs[2] + accs[6]
            out_v[r, pl.ds(48, 16)] = accs[3] + accs[7]

        gather(0, 0, sem0)

        def pair_body(i, carry):
            r0 = 2 * i
            gather(1, r0 + 1, sem1)
            drain(0, sem0)
            reduce_into(0, r0)

            @pl.when(r0 + 2 < B_PER_W)
            def _():
                gather(0, r0 + 2, sem0)

            drain(1, sem1)
            reduce_into(1, r0 + 1)
            return carry

        lax.fori_loop(0, B_PER_W // 2, pair_body, 0)
        pltpu.sync_copy(out_v, out_hbm.at[pl.ds(wid * B_PER_W, B_PER_W)])

    call = pl.kernel(
        body,
        out_type=jax.ShapeDtypeStruct((B, EMBED), jnp.float32),
        mesh=mesh,
        compiler_params=pltpu.CompilerParams(use_tc_tiling_on_sc=False),
        scratch_types=[
            pltpu.VMEM((B_PER_W, L), jnp.int32),
            pltpu.VMEM((2, L, EMBED), jnp.float32),
            pltpu.VMEM((B_PER_W, EMBED), jnp.float32),
            pltpu.SemaphoreType.DMA,
            pltpu.SemaphoreType.DMA,
        ],
    )
    return call(emb_table, idx)


def _tc_combine_body(idx_ref, sums_ref, pe_ref, wt_ref, b_ref, e1_ref, out_ref):
    idx = idx_ref[...]
    m = jnp.logical_and(idx != 0, idx != 1).astype(jnp.float32)
    n_eos = jnp.sum((idx == 1).astype(jnp.float32), axis=1, keepdims=True)
    count = jnp.sum(m, axis=1, keepdims=True)
    pe_sum = jnp.dot(m, pe_ref[...], preferred_element_type=jnp.float32)
    s_masked = sums_ref[...] - n_eos * e1_ref[...]
    pooled = (8.0 * s_masked + pe_sum) / (count + 1e-6)
    out_ref[...] = (
        jnp.dot(pooled, wt_ref[...], preferred_element_type=jnp.float32)
        + b_ref[...])


def _tc_combine(idx, sums, wt, bvec, e1):
    return pl.pallas_call(
        _tc_combine_body,
        out_shape=jax.ShapeDtypeStruct((B, OUT_DIM), jnp.float32),
    )(idx, sums, jnp.asarray(_PE), wt, bvec, e1)


def kernel(src_tok_idxs, emb_table, W, b):
    idx = src_tok_idxs.astype(jnp.int32)
    sums = _sc_gather_sum(emb_table, idx)
    return _tc_combine(idx, sums, W.T, b.reshape(1, OUT_DIM),
                       emb_table[1:2])
